# baseline (speedup 1.0000x reference)
import functools

import jax
import jax.numpy as jnp
from jax import lax
from jax.experimental import pallas as pl
from jax.experimental.pallas import tpu as pltpu
from jax.experimental.pallas import tpu_sc as plsc

_S = 64
_K = 8
_RAD2 = 0.05 * 0.05
_P = 5000
_PP = 5120
_L = 16
_NC = 2
_NS = 16
_NT = _NC * _NS
_RPT = _S // _NT
_SENT = 0x7FFFFFFF
_ZBIAS = 0x3F80


def _sc_body(px_hbm, py_hbm, pz_hbm, scal_hbm, idx_hbm, z_hbm, d_hbm,
             ptsx, ptsy, ptsz, scalv,
             cx0, cd0, ck0, cx1, cd1, ck1,
             oidx, oz, od):
    wid = lax.axis_index("s") * _NC + lax.axis_index("c")
    row0 = wid * _RPT

    pltpu.sync_copy(px_hbm, ptsx)
    pltpu.sync_copy(py_hbm, ptsy)
    pltpu.sync_copy(pz_hbm, ptsz)
    pltpu.sync_copy(scal_hbm, scalv)

    lanes = lax.broadcasted_iota(jnp.int32, (_L,), 0)

    def _splat(k):
        return scalv[pl.ds(k * _L, _L)]

    def _b(v):
        b = plsc.bitcast(v, jnp.int32)
        rnd = b + 0x7FFF + lax.bitwise_and(
            lax.shift_right_logical(b, 16), jnp.int32(1))
        rnd = lax.bitwise_and(rnd, jnp.int32(-65536))
        return plsc.bitcast(rnd, jnp.float32)

    r00, r10, r20 = _b(_splat(0)), _b(_splat(3)), _b(_splat(6))
    r01, r11, r21 = _b(_splat(1)), _b(_splat(4)), _b(_splat(7))
    r02, r12, r22 = _b(_splat(2)), _b(_splat(5)), _b(_splat(8))
    t0, t1, t2 = _splat(9), _splat(10), _splat(11)
    fv = _splat(12)

    cand = ((cx0, cd0, ck0), (cx1, cd1, ck1))

    def p1_body(i, cnts):
        sl = pl.ds(i * _L, _L)
        px = _b(ptsx[sl])
        py = _b(ptsy[sl])
        pz = _b(ptsz[sl])
        xv = px * r00 + py * r10 + pz * r20 + t0
        yv = px * r01 + py * r11 + pz * r21 + t1
        zv = px * r02 + py * r12 + pz * r22 + t2
        eps = jnp.float32(1e-8)
        denom = jnp.where(jnp.abs(zv) < eps, eps, zv)
        xn = fv * xv / denom
        yn = fv * yv / denom
        pidx = i * _L + lanes
        zbits = lax.shift_right_arithmetic(plsc.bitcast(zv, jnp.int32), 16)
        key = lax.bitwise_or(
            lax.shift_left(zbits - _ZBIAS, 13), pidx)
        base_ok = (zv > 0.0) & (pidx < _P)
        new_cnts = []
        for r in range(_RPT):
            gy = 1.0 - (2.0 * (row0 + r).astype(jnp.float32) + 1.0) * (1.0 / _S)
            dy = yn - gy
            dy2 = dy * dy
            m = (dy2 <= _RAD2) & base_ok
            mi = m.astype(jnp.int32)
            pos = cnts[r] + plsc.cumsum(mi) - mi
            cxr, cdr, ckr = cand[r]
            plsc.store_scatter(cxr, [pos], xn, mask=m)
            plsc.store_scatter(cdr, [pos], dy2, mask=m)
            plsc.store_scatter(ckr, [pos], key, mask=m)
            new_cnts.append(cnts[r] + jnp.sum(mi))
        return tuple(new_cnts)

    cnts = lax.fori_loop(0, _PP // _L, p1_body,
                         tuple(jnp.int32(0) for _ in range(_RPT)))

    for r in range(_RPT):
        cxr, cdr, ckr = cand[r]
        n = cnts[r]
        for q in range(_S // _L):
            gx = 1.0 - (2.0 * (q * _L + lanes).astype(jnp.float32) + 1.0) * (1.0 / _S)

            def p2_fn(c, state, cxr=cxr, cdr=cdr, ckr=ckr, gx=gx):
                keys = list(state[:_K])
                ds = list(state[_K:])
                ci = jnp.full((_L,), c, jnp.int32)
                cxv = plsc.load_gather(cxr, [ci])
                cdv = plsc.load_gather(cdr, [ci])
                ckv = plsc.load_gather(ckr, [ci])
                dx = cxv - gx
                d2 = dx * dx + cdv
                hit = d2 <= _RAD2

                def _insert(args):
                    ins_k, ins_d = args[0], args[1]
                    kys = list(args[2:2 + _K])
                    dds = list(args[2 + _K:])
                    for k in range(_K):
                        old_k = kys[k]
                        old_d = dds[k]
                        sw = ins_k < old_k
                        kys[k] = jnp.where(sw, ins_k, old_k)
                        dds[k] = jnp.where(sw, ins_d, old_d)
                        ins_k = jnp.where(sw, old_k, ins_k)
                        ins_d = jnp.where(sw, old_d, ins_d)
                    return tuple(kys) + tuple(dds)

                def _skip(args):
                    return tuple(args[2:])

                ins_k = jnp.where(hit, ckv, _SENT)
                return lax.cond(jnp.any(hit), _insert, _skip,
                                (ins_k, d2) + tuple(keys) + tuple(ds))

            init = tuple(jnp.full((_L,), _SENT, jnp.int32)
                         for _ in range(_K)) + \
                   tuple(jnp.zeros((_L,), jnp.float32) for _ in range(_K))
            state = lax.fori_loop(0, n, p2_fn, init)

            lanepix = r * _S + q * _L + lanes
            for k in range(_K):
                kk = state[k]
                dd = state[_K + k]
                empty = kk == _SENT
                idxv = jnp.where(empty, -1,
                                 lax.bitwise_and(kk, jnp.int32(0x1FFF)))
                zrec = plsc.bitcast(
                    lax.shift_left(
                        lax.shift_right_arithmetic(kk, 13) + _ZBIAS, 16),
                    jnp.float32)
                zov = jnp.where(empty, jnp.float32(-1.0), zrec)
                dov = jnp.where(empty, jnp.float32(-1.0), dd)
                pos = lanepix * _K + k
                plsc.store_scatter(oidx, [pos], idxv)
                plsc.store_scatter(oz, [pos], zov)
                plsc.store_scatter(od, [pos], dov)

    base = wid * (_RPT * _S * _K)
    nout = _RPT * _S * _K
    pltpu.sync_copy(oidx, idx_hbm.at[pl.ds(base, nout)])
    pltpu.sync_copy(oz, z_hbm.at[pl.ds(base, nout)])
    pltpu.sync_copy(od, d_hbm.at[pl.ds(base, nout)])


def kernel(points, R, T, focal_length):
    N, P, _ = points.shape
    pts = points[0].T
    pts = jnp.pad(pts, ((0, 0), (0, _PP - P)))
    pxa, pya, pza = pts[0], pts[1], pts[2]
    scal = jnp.concatenate(
        [R[0].reshape(-1), T[0].reshape(-1),
         focal_length[:1].astype(jnp.float32)])
    scal = jnp.broadcast_to(scal[:, None], (13, _L)).reshape(-1)

    mesh = plsc.VectorSubcoreMesh(core_axis_name="c", subcore_axis_name="s")
    nel = _S * _S * _K
    run = pl.kernel(
        _sc_body,
        out_type=(
            jax.ShapeDtypeStruct((nel,), jnp.int32),
            jax.ShapeDtypeStruct((nel,), jnp.float32),
            jax.ShapeDtypeStruct((nel,), jnp.float32),
        ),
        mesh=mesh,
        scratch_types=[
            pltpu.VMEM((_PP,), jnp.float32),
            pltpu.VMEM((_PP,), jnp.float32),
            pltpu.VMEM((_PP,), jnp.float32),
            pltpu.VMEM((13 * _L,), jnp.float32),
            pltpu.VMEM((_PP,), jnp.float32),
            pltpu.VMEM((_PP,), jnp.float32),
            pltpu.VMEM((_PP,), jnp.int32),
            pltpu.VMEM((_PP,), jnp.float32),
            pltpu.VMEM((_PP,), jnp.float32),
            pltpu.VMEM((_PP,), jnp.int32),
            pltpu.VMEM((_RPT * _S * _K,), jnp.int32),
            pltpu.VMEM((_RPT * _S * _K,), jnp.float32),
            pltpu.VMEM((_RPT * _S * _K,), jnp.float32),
        ],
        compiler_params=pltpu.CompilerParams(needs_layout_passes=False),
    )
    idx, zb, db = run(pxa, pya, pza, scal)
    return (idx.reshape(1, _S, _S, _K),
            zb.reshape(1, _S, _S, _K),
            db.reshape(1, _S, _S, _K))


# speedup vs baseline: 1.2034x; 1.2034x over previous
import jax
import jax.numpy as jnp
from jax import lax
from jax.experimental import pallas as pl
from jax.experimental.pallas import tpu as pltpu
from jax.experimental.pallas import tpu_sc as plsc

_S = 64
_K = 8
_RAD2 = 0.05 * 0.05
_P = 5000
_PP = 5120
_L = 16
_NC = 2
_NS = 16
_NT = _NC * _NS
_RPT = _S // _NT
_SENT = 0x7FFFFFFF
_ZBIAS = 0x3F80


def _sc_body(px_hbm, py_hbm, pz_hbm, scal_hbm, idx_hbm, z_hbm, d_hbm,
             ptsx, ptsy, ptsz, scalv,
             cx0, cd0, ck0, cx1, cd1, ck1,
             oidx, oz, od):
    wid = lax.axis_index("s") * _NC + lax.axis_index("c")
    row0 = wid * _RPT

    pltpu.sync_copy(px_hbm, ptsx)
    pltpu.sync_copy(py_hbm, ptsy)
    pltpu.sync_copy(pz_hbm, ptsz)
    pltpu.sync_copy(scal_hbm, scalv)

    lanes = lax.broadcasted_iota(jnp.int32, (_L,), 0)

    def _splat(k):
        return scalv[pl.ds(k * _L, _L)]

    def _b(v):
        b = plsc.bitcast(v, jnp.int32)
        rnd = b + 0x7FFF + lax.bitwise_and(
            lax.shift_right_logical(b, 16), jnp.int32(1))
        rnd = lax.bitwise_and(rnd, jnp.int32(-65536))
        return plsc.bitcast(rnd, jnp.float32)

    r00, r10, r20 = _b(_splat(0)), _b(_splat(3)), _b(_splat(6))
    r01, r11, r21 = _b(_splat(1)), _b(_splat(4)), _b(_splat(7))
    r02, r12, r22 = _b(_splat(2)), _b(_splat(5)), _b(_splat(8))
    t0, t1, t2 = _splat(9), _splat(10), _splat(11)
    fv = _splat(12)

    cand = ((cx0, cd0, ck0), (cx1, cd1, ck1))

    def p1_body(i, cnts):
        sl = pl.ds(i * _L, _L)
        px = _b(ptsx[sl])
        py = _b(ptsy[sl])
        pz = _b(ptsz[sl])
        xv = px * r00 + py * r10 + pz * r20 + t0
        yv = px * r01 + py * r11 + pz * r21 + t1
        zv = px * r02 + py * r12 + pz * r22 + t2
        eps = jnp.float32(1e-8)
        denom = jnp.where(jnp.abs(zv) < eps, eps, zv)
        xn = fv * xv / denom
        yn = fv * yv / denom
        pidx = i * _L + lanes
        zbits = lax.shift_right_arithmetic(plsc.bitcast(zv, jnp.int32), 16)
        key = lax.bitwise_or(
            lax.shift_left(zbits - _ZBIAS, 13), pidx)
        base_ok = (zv > 0.0) & (pidx < _P)
        new_cnts = []
        for r in range(_RPT):
            gy = 1.0 - (2.0 * (row0 + r).astype(jnp.float32) + 1.0) * (1.0 / _S)
            dy = yn - gy
            dy2 = dy * dy
            m = (dy2 <= _RAD2) & base_ok
            mi = m.astype(jnp.int32)
            pos = cnts[r] + plsc.cumsum(mi) - mi
            cxr, cdr, ckr = cand[r]
            plsc.store_scatter(cxr, [pos], xn, mask=m)
            plsc.store_scatter(cdr, [pos], dy2, mask=m)
            plsc.store_scatter(ckr, [pos], key, mask=m)
            new_cnts.append(cnts[r] + jnp.sum(mi))
        return tuple(new_cnts)

    cnts = lax.fori_loop(0, _PP // _L, p1_body,
                         tuple(jnp.int32(0) for _ in range(_RPT)))

    for r in range(_RPT):
        cxr, cdr, ckr = cand[r]
        n = cnts[r]
        for q in range(_S // _L):
            gx = 1.0 - (2.0 * (q * _L + lanes).astype(jnp.float32) + 1.0) * (1.0 / _S)

            def p2_fn(c, state, cxr=cxr, cdr=cdr, ckr=ckr, gx=gx):
                keys = list(state[:_K])
                ds = list(state[_K:])
                ci = jnp.full((_L,), c, jnp.int32)
                cxv = plsc.load_gather(cxr, [ci])
                cdv = plsc.load_gather(cdr, [ci])
                ckv = plsc.load_gather(ckr, [ci])
                dx = cxv - gx
                d2 = dx * dx + cdv
                hit = d2 <= _RAD2
                ins_k = jnp.where(hit, ckv, _SENT)
                ins_d = d2
                for k in range(_K):
                    old_k = keys[k]
                    old_d = ds[k]
                    sw = ins_k < old_k
                    keys[k] = jnp.where(sw, ins_k, old_k)
                    ds[k] = jnp.where(sw, ins_d, old_d)
                    ins_k = jnp.where(sw, old_k, ins_k)
                    ins_d = jnp.where(sw, old_d, ins_d)
                return tuple(keys) + tuple(ds)

            init = tuple(jnp.full((_L,), _SENT, jnp.int32)
                         for _ in range(_K)) + \
                   tuple(jnp.zeros((_L,), jnp.float32) for _ in range(_K))
            state = lax.fori_loop(0, n, p2_fn, init)

            lanepix = r * _S + q * _L + lanes
            for k in range(_K):
                kk = state[k]
                dd = state[_K + k]
                empty = kk == _SENT
                idxv = jnp.where(empty, -1,
                                 lax.bitwise_and(kk, jnp.int32(0x1FFF)))
                zrec = plsc.bitcast(
                    lax.shift_left(
                        lax.shift_right_arithmetic(kk, 13) + _ZBIAS, 16),
                    jnp.float32)
                zov = jnp.where(empty, jnp.float32(-1.0), zrec)
                dov = jnp.where(empty, jnp.float32(-1.0), dd)
                pos = lanepix * _K + k
                plsc.store_scatter(oidx, [pos], idxv)
                plsc.store_scatter(oz, [pos], zov)
                plsc.store_scatter(od, [pos], dov)

    base = wid * (_RPT * _S * _K)
    nout = _RPT * _S * _K
    pltpu.sync_copy(oidx, idx_hbm.at[pl.ds(base, nout)])
    pltpu.sync_copy(oz, z_hbm.at[pl.ds(base, nout)])
    pltpu.sync_copy(od, d_hbm.at[pl.ds(base, nout)])


def kernel(points, R, T, focal_length):
    N, P, _ = points.shape
    pts = points[0].T
    pts = jnp.pad(pts, ((0, 0), (0, _PP - P)))
    pxa, pya, pza = pts[0], pts[1], pts[2]
    scal = jnp.concatenate(
        [R[0].reshape(-1), T[0].reshape(-1),
         focal_length[:1].astype(jnp.float32)])
    scal = jnp.broadcast_to(scal[:, None], (13, _L)).reshape(-1)

    mesh = plsc.VectorSubcoreMesh(core_axis_name="c", subcore_axis_name="s")
    nel = _S * _S * _K
    run = pl.kernel(
        _sc_body,
        out_type=(
            jax.ShapeDtypeStruct((nel,), jnp.int32),
            jax.ShapeDtypeStruct((nel,), jnp.float32),
            jax.ShapeDtypeStruct((nel,), jnp.float32),
        ),
        mesh=mesh,
        scratch_types=[
            pltpu.VMEM((_PP,), jnp.float32),
            pltpu.VMEM((_PP,), jnp.float32),
            pltpu.VMEM((_PP,), jnp.float32),
            pltpu.VMEM((13 * _L,), jnp.float32),
            pltpu.VMEM((_PP,), jnp.float32),
            pltpu.VMEM((_PP,), jnp.float32),
            pltpu.VMEM((_PP,), jnp.int32),
            pltpu.VMEM((_PP,), jnp.float32),
            pltpu.VMEM((_PP,), jnp.float32),
            pltpu.VMEM((_PP,), jnp.int32),
            pltpu.VMEM((_RPT * _S * _K,), jnp.int32),
            pltpu.VMEM((_RPT * _S * _K,), jnp.float32),
            pltpu.VMEM((_RPT * _S * _K,), jnp.float32),
        ],
        compiler_params=pltpu.CompilerParams(needs_layout_passes=False),
    )
    idx, zb, db = run(pxa, pya, pza, scal)
    return (idx.reshape(1, _S, _S, _K),
            zb.reshape(1, _S, _S, _K),
            db.reshape(1, _S, _S, _K))
